# baseline (speedup 1.0000x reference)
import jax
import jax.numpy as jnp
from jax.experimental import pallas as pl
from jax.experimental.pallas import tpu as pltpu

_BLOCK_ROWS = 256
_LANES = 128


def _body(x_ref, o_ref):
    x = x_ref[...]
    shape = x.shape
    p = jax.lax.broadcasted_iota(jnp.int32, shape, 2) & 3
    right1 = pltpu.roll(x, _LANES - 1, 2)
    left1 = pltpu.roll(x, 1, 2)
    s1 = jnp.where((p & 1) == 0, right1, left1)
    mx = jnp.maximum(x, s1)
    mn = jnp.minimum(x, s1)
    lo = p < 2
    mx_sw = jnp.where(lo, pltpu.roll(mx, _LANES - 2, 2), pltpu.roll(mx, 2, 2))
    mn_sw = jnp.where(lo, pltpu.roll(mn, _LANES - 2, 2), pltpu.roll(mn, 2, 2))
    second = jnp.maximum(jnp.minimum(mx, mx_sw), jnp.maximum(mn, mn_sw))
    o_ref[...] = jnp.where(x >= second, x, jnp.zeros_like(x))


def kernel(input):
    n, d = input.shape
    sub = d // _LANES
    x3 = input.reshape(n, sub, _LANES)
    grid = n // _BLOCK_ROWS
    out = pl.pallas_call(
        _body,
        grid=(grid,),
        in_specs=[pl.BlockSpec((_BLOCK_ROWS, sub, _LANES), lambda i: (i, 0, 0))],
        out_specs=pl.BlockSpec((_BLOCK_ROWS, sub, _LANES), lambda i: (i, 0, 0)),
        out_shape=jax.ShapeDtypeStruct((n, sub, _LANES), input.dtype),
        compiler_params=pltpu.CompilerParams(
            dimension_semantics=("arbitrary",),
        ),
    )(x3)
    return out.reshape(n, d)
